# baseline (speedup 1.0000x reference)
import functools

import jax
import jax.numpy as jnp
from jax import lax
from jax.experimental import pallas as pl
from jax.experimental.pallas import tpu as pltpu
from jax.experimental.pallas import tpu_sc as plsc

B = 16384
N = 1000
NP = 1024
R = 64
NC = 2
NS = 16
NW = NC * NS
BPW = B // NW
CH = 128
NCH = BPW // CH
L = 16


def _mm_body(t0t_ref, t1_ref, g_ref):
    t0t = t0t_ref[...]
    for t in range(NP // 128):
        lo = t * 128
        hi = min(lo + 128, N)
        d = lax.dot_general(
            t0t, t1_ref[:, lo:hi],
            dimension_numbers=(((0,), (0,)), ((), ())),
            preferred_element_type=jnp.float32,
            precision=lax.Precision.DEFAULT,
        )
        if hi - lo < 128:
            d = jnp.pad(d, ((0, 0), (0, 128 - (hi - lo))))
        g_ref[t] = d


def _make_gather_kernel():
    mesh = plsc.VectorSubcoreMesh(core_axis_name="c", subcore_axis_name="s")

    @functools.partial(
        pl.kernel,
        mesh=mesh,
        out_type=jax.ShapeDtypeStruct((B,), jnp.float32),
        scratch_types=[
            pltpu.VMEM((2, BPW), jnp.int32),
            pltpu.VMEM((BPW,), jnp.int32),
            pltpu.VMEM((BPW,), jnp.float32),
            pltpu.SemaphoreType.DMA,
        ],
    )
    def k(idxt_hbm, g_hbm, out_hbm, i01_v, f_v, o_v, sem):
        wid = lax.axis_index("s") * NC + lax.axis_index("c")

        pltpu.sync_copy(idxt_hbm.at[:, pl.ds(wid * BPW, BPW)], i01_v)

        copies = []
        for c in range(NCH):
            for vi in range(CH // L):
                s = pl.ds(c * CH + vi * L, L)
                i = i01_v[0, s]
                j = i01_v[1, s]
                f_v[s] = (lax.shift_right_logical(j, 7) * (N * 128)
                          + i * 128 + lax.bitwise_and(j, 127))
            copies.append(
                pltpu.async_copy(g_hbm.at[f_v.at[pl.ds(c * CH, CH)]],
                                 o_v.at[pl.ds(c * CH, CH)], sem))
        for cp in copies:
            cp.wait()

        pltpu.sync_copy(o_v, out_hbm.at[pl.ds(wid * BPW, BPW)])

    return k


_gather_kernel = _make_gather_kernel()


def kernel(indices, C0, C1):
    idxt = jnp.transpose(indices.astype(jnp.int32))
    t0t = jnp.transpose(C0[0])

    g = pl.pallas_call(
        _mm_body,
        out_shape=jax.ShapeDtypeStruct((NP // 128, N, 128), jnp.float32),
    )(t0t, C1[:, :, 0])

    return _gather_kernel(idxt, g.reshape(NP * N))
